# Initial kernel scaffold; baseline (speedup 1.0000x reference)
import jax
import jax.numpy as jnp
from jax.experimental import pallas as pl


def kernel(x, W1, b1, W2, b2):
    raise NotImplementedError("write your pallas kernel here")


# speedup vs baseline: 1.0085x; 1.0085x over previous
import functools

import jax
import jax.numpy as jnp
from jax.experimental import pallas as pl
from jax.experimental.pallas import tpu as pltpu

_LBW = 0.01
_KT = 256


def _gate_kernel(x_ref, w1_ref, b1_ref, w2_ref, b2_ref,
                 wts_ref, idx_ref, loss_ref, acc_ref, *, nk, s, e, topk):
    k = pl.program_id(0)

    @pl.when(k == 0)
    def _init():
        acc_ref[...] = jnp.zeros_like(acc_ref)

    r = jnp.sum(x_ref[...], axis=1) * (1.0 / s)
    acc_ref[...] += jnp.dot(r, w1_ref[...],
                            preferred_element_type=jnp.float32)

    @pl.when(k == nk - 1)
    def _epilogue():
        h = acc_ref[...] + b1_ref[...]
        h = h * jax.nn.sigmoid(h)
        logits = jnp.dot(h, w2_ref[...],
                         preferred_element_type=jnp.float32) + b2_ref[...]
        b = logits.shape[0]
        iota_e = jax.lax.broadcasted_iota(jnp.int32, (b, e), 1)
        m1 = jnp.max(logits, axis=-1)
        i1 = jnp.argmax(logits, axis=-1).astype(jnp.int32)
        masked = jnp.where(iota_e == i1[:, None], -jnp.inf, logits)
        m2 = jnp.max(masked, axis=-1)
        i2 = jnp.argmax(masked, axis=-1).astype(jnp.int32)
        e2 = jnp.exp(m2 - m1)
        denom = 1.0 + e2
        w_first = 1.0 / denom
        w_second = e2 / denom
        iota2 = jax.lax.broadcasted_iota(jnp.int32, (b, topk), 1)
        wts_ref[...] = jnp.where(iota2 == 0, w_first[:, None], w_second[:, None])
        idx_ref[...] = jnp.where(iota2 == 0, i1[:, None], i2[:, None])
        probs = jax.nn.softmax(logits, axis=-1)
        mean_prob = jnp.mean(probs, axis=0)
        onehot = ((iota_e == i1[:, None]).astype(jnp.float32)
                  + (iota_e == i2[:, None]).astype(jnp.float32))
        usage = jnp.sum(onehot, axis=0)
        mean_usage = usage / (b * topk)
        loss = _LBW * e * jnp.sum(mean_prob * mean_usage)
        loss_ref[...] = loss[None, None]


def kernel(x, W1, b1, W2, b2):
    B, S, D = x.shape
    E = W2.shape[1]
    TOPK = 2
    nk = D // _KT

    grid = (nk,)
    kfn = functools.partial(_gate_kernel, nk=nk, s=S, e=E, topk=TOPK)
    wts, idx, loss = pl.pallas_call(
        kfn,
        grid=grid,
        in_specs=[
            pl.BlockSpec((B, S, _KT), lambda k: (0, 0, k)),
            pl.BlockSpec((_KT, D), lambda k: (k, 0)),
            pl.BlockSpec((1, D), lambda k: (0, 0)),
            pl.BlockSpec((D, E), lambda k: (0, 0)),
            pl.BlockSpec((1, E), lambda k: (0, 0)),
        ],
        out_specs=[
            pl.BlockSpec((B, TOPK), lambda k: (0, 0)),
            pl.BlockSpec((B, TOPK), lambda k: (0, 0)),
            pl.BlockSpec((1, 1), lambda k: (0, 0)),
        ],
        out_shape=[
            jax.ShapeDtypeStruct((B, TOPK), jnp.float32),
            jax.ShapeDtypeStruct((B, TOPK), jnp.int32),
            jax.ShapeDtypeStruct((1, 1), jnp.float32),
        ],
        scratch_shapes=[pltpu.VMEM((B, D), jnp.float32)],
        compiler_params=pltpu.CompilerParams(
            dimension_semantics=("arbitrary",),
        ),
    )(x, W1, b1.reshape(1, D), W2, b2.reshape(1, E))
    return wts, idx, loss.reshape(())
